# baseline (speedup 1.0000x reference)
import functools

import jax
import jax.numpy as jnp
from jax import lax
from jax.experimental import pallas as pl
from jax.experimental.pallas import tpu as pltpu
from jax.experimental.pallas import tpu_sc as plsc

NUM_ENTITIES = 1000000
EMB_DIM = 32
BATCH = 16384

NUM_CORES = 2
NUM_SUBCORES = 16
LANES = 16
NUM_WORKERS = NUM_CORES * NUM_SUBCORES
B_PER_W = BATCH // NUM_WORKERS
CHUNK = 128
NUM_CHUNKS = B_PER_W // CHUNK
GROUPS = B_PER_W // LANES

_mesh = plsc.VectorSubcoreMesh(core_axis_name="c", subcore_axis_name="s")


@functools.partial(
    pl.kernel,
    mesh=_mesh,
    out_type=[
        jax.ShapeDtypeStruct((BATCH, EMB_DIM), jnp.float32),
        jax.ShapeDtypeStruct((BATCH,), jnp.float32),
    ],
    scratch_types=[
        pltpu.VMEM((NUM_CHUNKS, CHUNK), jnp.int32),
        pltpu.VMEM((B_PER_W, EMB_DIM), jnp.float32),
        pltpu.VMEM((B_PER_W,), jnp.float32),
        pltpu.SemaphoreType.DMA,
    ],
    compiler_params=pltpu.CompilerParams(
        needs_layout_passes=False, use_tc_tiling_on_sc=False
    ),
)
def _emb_kernel(ents_hbm, table_hbm, out_hbm, norm_hbm, idx_v, rows_v, norms_v, sem):
    wid = lax.axis_index("s") * NUM_CORES + lax.axis_index("c")
    base = wid * B_PER_W

    for i in range(NUM_CHUNKS):
        pltpu.sync_copy(ents_hbm.at[pl.ds(base + i * CHUNK, CHUNK)], idx_v.at[i])
    copies = []
    for i in range(NUM_CHUNKS):
        copies.append(
            pltpu.async_copy(
                table_hbm.at[idx_v.at[i]],
                rows_v.at[pl.ds(i * CHUNK, CHUNK)],
                sem,
            )
        )
    for cp in copies:
        cp.wait()

    def sqrt16(x):
        i = plsc.bitcast(x, jnp.int32)
        i = jnp.int32(0x5F3759DF) - (i >> 1)
        y = plsc.bitcast(i, jnp.float32)
        for _ in range(4):
            y = y * (1.5 - 0.5 * x * y * y)
        return jnp.where(x > 0.0, x * y, 0.0)

    lane_iota = jax.lax.iota(jnp.int32, LANES)

    def group_body(g, _):
        row_idx = g * LANES + lane_iota
        acc = jnp.zeros((LANES,), jnp.float32)
        for c in range(EMB_DIM):
            col_idx = jnp.full((LANES,), c, jnp.int32)
            v = plsc.load_gather(rows_v, [row_idx, col_idx])
            acc = acc + v * v
        norms_v[pl.ds(g * LANES, LANES)] = sqrt16(acc)
        return 0

    lax.fori_loop(0, GROUPS, group_body, 0)

    pltpu.sync_copy(rows_v, out_hbm.at[pl.ds(base, B_PER_W)])
    pltpu.sync_copy(norms_v, norm_hbm.at[pl.ds(base, B_PER_W)])


def kernel(ents, W_means):
    rows, norms = _emb_kernel(ents, W_means)
    return (rows, norms)
